# baseline (speedup 1.0000x reference)
import functools

import jax
import jax.numpy as jnp
from jax import lax
from jax.experimental import pallas as pl
from jax.experimental.pallas import tpu as pltpu
from jax.experimental.pallas import tpu_sc as plsc

_NW = 32
_LANES = 16


def _sc_mesh():
    return plsc.VectorSubcoreMesh(
        core_axis_name="c", subcore_axis_name="s", num_cores=2, num_subcores=16
    )


def _wid():
    return lax.axis_index("s") * 2 + lax.axis_index("c")


def _embed_body(f_ref, wx_ref, wn_ref, rx_ref, pn_ref, pna_ref, pnb_ref):
    f = f_ref[...]
    rx_ref[...] = jnp.maximum(
        jnp.dot(f, wx_ref[...], preferred_element_type=jnp.float32), 0.0
    )
    pn = jnp.dot(f, wn_ref[...], preferred_element_type=jnp.float32)
    pn_ref[...] = pn
    pnbf = pn.astype(jnp.bfloat16)
    pna_ref[...] = pnbf
    pnb_ref[...] = pnbf


def _embed(features, W1x, W1n):
    n, d = features.shape
    h = W1x.shape[1]
    rb = 2000
    assert n % rb == 0
    out = pl.pallas_call(
        _embed_body,
        grid=(n // rb,),
        in_specs=[
            pl.BlockSpec((rb, d), lambda i: (i, 0)),
            pl.BlockSpec((d, h), lambda i: (0, 0)),
            pl.BlockSpec((d, h), lambda i: (0, 0)),
        ],
        out_specs=[
            pl.BlockSpec((rb, h), lambda i: (i, 0)),
            pl.BlockSpec((rb, h), lambda i: (i, 0)),
            pl.BlockSpec((rb, h), lambda i: (i, 0)),
            pl.BlockSpec((rb, h), lambda i: (i, 0)),
        ],
        out_shape=[
            jax.ShapeDtypeStruct((n, h), jnp.float32),
            jax.ShapeDtypeStruct((n, h), jnp.float32),
            jax.ShapeDtypeStruct((n, h), jnp.bfloat16),
            jax.ShapeDtypeStruct((n, h), jnp.bfloat16),
        ],
    )(features, W1x, W1n)
    return out


def _mean10(PN, PNb, nbr10):
    hi32 = PN.shape[1]
    h = hi32 * 2
    npad = nbr10.shape[0] * nbr10.shape[1] // 10
    g = 8
    nbuf = 4
    npw0 = 384
    npw1 = (npad - 16 * npw0) // 16
    nblk0, nblk1 = npw0 // g, npw1 // g
    assert nblk0 % nbuf == 0 and nblk1 % nbuf == 0
    nblk_max = max(nblk0, nblk1)

    @functools.partial(
        pl.kernel,
        mesh=_sc_mesh(),
        out_type=jax.ShapeDtypeStruct((npad, h), jnp.float32),
        scratch_types=[
            pltpu.VMEM((nblk_max, g * 10), jnp.int32),
            [pltpu.VMEM((g * 10, hi32), jnp.int32)] * nbuf,
            [pltpu.VMEM((g, h), jnp.float32)] * nbuf,
            [pltpu.SemaphoreType.DMA] * nbuf,
            [pltpu.SemaphoreType.DMA] * nbuf,
        ],
    )
    def k(pn_hbm, pnb_hbm, idx_hbm, out_hbm, idxall_v, rows_v, acc_v, gsem, wsem):
        core = lax.axis_index("c")
        sid = lax.axis_index("s")
        w0 = pl.multiple_of(
            jnp.where(core == 0, sid * npw0, 16 * npw0 + sid * npw1), 8)
        nblk_w = jnp.where(core == 0, nblk0, nblk1)
        nouter_w = jnp.where(core == 0, nblk0 // nbuf, nblk1 // nbuf)

        def gather(q, t):
            return pltpu.make_async_copy(
                pn_hbm.at[idxall_v.at[q]], rows_v[t], gsem[t])

        def gather_start(q, t):
            pl.when(core == 0)(lambda: gather(q, t).start())
            pl.when(core == 1)(lambda: pltpu.make_async_copy(
                pnb_hbm.at[idxall_v.at[q]], rows_v[t], gsem[t]).start())

        def wback(q, t):
            return pltpu.make_async_copy(
                acc_v[t], out_hbm.at[pl.ds(w0 + q * g, g)], wsem[t])

        pl.when(core == 0)(lambda: pltpu.sync_copy(
            idx_hbm.at[pl.ds(sid * nblk0, nblk0)], idxall_v))
        pl.when(core == 1)(lambda: pltpu.sync_copy(
            idx_hbm.at[pl.ds(16 * nblk0 + sid * nblk1, nblk1)],
            idxall_v.at[pl.ds(0, nblk1)]))
        for t in range(nbuf):
            gather_start(t, t)

        def outer(bb, carry):
            for t in range(nbuf):
                q = bb * nbuf + t
                gather(q, t).wait()
                pl.when(q >= nbuf)(lambda: wback(q - nbuf, t).wait())

                def node(i, carry2):
                    r0 = i * 10
                    for c in range(hi32 // _LANES):
                        s = pl.ds(c * _LANES, _LANES)
                        a = jnp.zeros((_LANES,), jnp.float32)
                        b2 = jnp.zeros((_LANES,), jnp.float32)
                        for kk in range(10):
                            v = rows_v[t][r0 + kk, s]
                            a = a + lax.bitcast_convert_type(
                                lax.shift_left(v, 16), jnp.float32)
                            b2 = b2 + lax.bitcast_convert_type(
                                v & jnp.int32(-65536), jnp.float32)
                        acc_v[t][i, pl.ds(c * 32, 16)] = jnp.maximum(a * 0.1, 0.0)
                        acc_v[t][i, pl.ds(c * 32 + 16, 16)] = jnp.maximum(
                            b2 * 0.1, 0.0)
                    return carry2

                lax.fori_loop(0, g, node, 0)
                wback(q, t).start()
                pl.when(q + nbuf < nblk_w)(lambda: gather_start(q + nbuf, t))
            return carry

        lax.fori_loop(0, nouter_w, outer, 0)
        for t in range(nbuf):
            wback(nblk_w - nbuf + t, t).wait()

    return k(PN, PNb, nbr10)


def _seeds(RX, PN, RM, ids, nbr25p):
    h = RX.shape[1]
    b = ids.shape[0]
    spw = b // _NW
    g = 4
    nblk = spw // g
    rows = g * 25
    stride = 104
    nc = h // _LANES
    assert nbr25p.shape == (_NW * nblk, stride)

    @functools.partial(
        pl.kernel,
        mesh=_sc_mesh(),
        out_type=[jax.ShapeDtypeStruct((b, h), jnp.float32)] * 4,
        scratch_types=[
            pltpu.VMEM((spw,), jnp.int32),
            pltpu.VMEM((nblk, stride), jnp.int32),
            [pltpu.VMEM((stride, h), jnp.float32)] * 3,
            pltpu.VMEM((spw, h), jnp.float32),
            [pltpu.VMEM((g, h), jnp.float32)] * 3,
            [pltpu.SemaphoreType.DMA] * 3,
            [pltpu.SemaphoreType.DMA] * 3,
            pltpu.SemaphoreType.DMA,
        ],
    )
    def k(rx_hbm, pn_hbm, rm_hbm, ids_hbm, nbr_hbm,
          x0_hbm, m25_hbm, aga_hbm, agb_hbm,
          ids_v, idxall_v, rows_v, x0_v, acc_v, gsem, wsem, xsem):
        s0 = _wid() * spw
        tables = [pn_hbm, rx_hbm, rm_hbm]
        outs = [m25_hbm, aga_hbm, agb_hbm]
        relus = [True, False, False]

        def gather(bb, t):
            return pltpu.make_async_copy(
                tables[t].at[idxall_v.at[bb]], rows_v[t], gsem[t])

        def wback(bb, t):
            return pltpu.make_async_copy(
                acc_v[t], outs[t].at[pl.ds(s0 + bb * g, g)], wsem[t])

        pltpu.sync_copy(ids_hbm.at[pl.ds(s0, spw)], ids_v)
        x0copy = pltpu.make_async_copy(rx_hbm.at[ids_v], x0_v, xsem)
        x0copy.start()
        pltpu.sync_copy(nbr_hbm.at[pl.ds(_wid() * nblk, nblk)], idxall_v)
        for t in range(3):
            gather(0, t).start()

        def blk(bb, carry):
            for t in range(3):
                gather(bb, t).wait()
                pl.when(bb >= 1)(lambda: wback(bb - 1, t).wait())

                def node(i, carry2):
                    r0 = i * 25
                    for c in range(nc):
                        s = pl.ds(c * _LANES, _LANES)
                        a = rows_v[t][r0, s]
                        for kk in range(1, 25):
                            a = a + rows_v[t][r0 + kk, s]
                        a = a * (1.0 / 25.0)
                        if relus[t]:
                            a = jnp.maximum(a, 0.0)
                        acc_v[t][i, s] = a
                    return carry2

                lax.fori_loop(0, g, node, 0)
                wback(bb, t).start()
                pl.when(bb + 1 < nblk)(lambda: gather(bb + 1, t).start())
            return carry

        lax.fori_loop(0, nblk, blk, 0)
        x0copy.wait()
        pltpu.sync_copy(x0_v, x0_hbm.at[pl.ds(s0, spw)])
        for t in range(3):
            wback(nblk - 1, t).wait()

    return k(RX, PN, RM, ids, nbr25p)


def _head_body(x0a_ref, x0b_ref, a2a_ref, a2b_ref, w2x_ref, w2n_ref,
               fcw_ref, fcb_ref, o_ref):
    x0 = jnp.concatenate([x0a_ref[...], x0b_ref[...]], axis=1)
    a2 = jnp.concatenate([a2a_ref[...], a2b_ref[...]], axis=1)
    hx = jnp.dot(x0, w2x_ref[...], preferred_element_type=jnp.float32)
    hn = jnp.dot(a2, w2n_ref[...], preferred_element_type=jnp.float32)
    hcat = jnp.concatenate([hx, hn], axis=1)
    ss = jnp.sum(hcat * hcat, axis=1, keepdims=True)
    nrm = jnp.maximum(jnp.sqrt(ss), 1e-12)
    hcat = hcat / nrm
    o_ref[...] = (
        jnp.dot(hcat, fcw_ref[...], preferred_element_type=jnp.float32)
        + fcb_ref[...]
    )


def _head(X0, RM25, AGa, AGb, W2x, W2n, fcW, fcb):
    b = X0.shape[0]
    ncls = fcW.shape[1]
    return pl.pallas_call(
        _head_body,
        out_shape=jax.ShapeDtypeStruct((b, ncls), jnp.float32),
    )(X0, RM25, AGa, AGb, W2x, W2n, fcW, fcb.reshape(1, -1))


def kernel(ids, features, adj, W1x, W1n, W2x, W2n, fcW, fcb):
    ids = ids.astype(jnp.int32)
    adj = adj.astype(jnp.int32)
    n = features.shape[0]
    npad = ((n + 8 * _NW - 1) // (8 * _NW)) * (8 * _NW)

    RX, PN, PNbfa, PNbfb = _embed(features, W1x, W1n)
    h1 = PN.shape[1]
    PNia = lax.bitcast_convert_type(
        PNbfa.reshape(n, h1 // 2, 2), jnp.int32)
    PNib = lax.bitcast_convert_type(
        PNbfb.reshape(n, h1 // 2, 2), jnp.int32)
    nbr10 = jnp.pad(adj[:, :10], ((0, npad - n), (0, 0))).reshape(-1, 80)
    RM = _mean10(PNia, PNib, nbr10)
    nbr25 = jnp.take(adj, ids, axis=0)[:, :25].reshape(-1, 100)
    nbr25p = jnp.pad(nbr25, ((0, 0), (0, 4)))
    X0, RM25, AGa, AGb = _seeds(RX, PN, RM, ids, nbr25p)
    h2 = RM.shape[1]
    blk = (jnp.arange(h2) // 32) * 32
    j = jnp.arange(h2) % 32
    perm = blk + jnp.where(j < 16, 2 * j, 2 * (j - 16) + 1)
    W2n_adj = jnp.concatenate([W2n[:h2], W2n[h2:][perm]], axis=0)
    return _head(X0, RM25, AGa, AGb, W2x, W2n_adj, fcW, fcb)


# speedup vs baseline: 1.8627x; 1.8627x over previous
import functools

import jax
import jax.numpy as jnp
from jax import lax
from jax.experimental import pallas as pl
from jax.experimental.pallas import tpu as pltpu
from jax.experimental.pallas import tpu_sc as plsc

_NW = 32
_LANES = 16


def _sc_mesh():
    return plsc.VectorSubcoreMesh(
        core_axis_name="c", subcore_axis_name="s", num_cores=2, num_subcores=16
    )


def _wid():
    return lax.axis_index("s") * 2 + lax.axis_index("c")


def _embed_body(f_ref, wx_ref, wn_ref, rx_ref, pn_ref, pna_ref, pnb_ref):
    f = f_ref[...]
    rx_ref[...] = jnp.maximum(
        jnp.dot(f, wx_ref[...], preferred_element_type=jnp.float32), 0.0
    )
    pn = jnp.dot(f, wn_ref[...], preferred_element_type=jnp.float32)
    pn_ref[...] = pn
    h2 = pn.shape[1] // 2
    lo = lax.bitcast_convert_type(
        pn[:, :h2].astype(jnp.bfloat16).astype(jnp.float32), jnp.int32)
    hi = lax.bitcast_convert_type(
        pn[:, h2:].astype(jnp.bfloat16).astype(jnp.float32), jnp.int32)
    packed = lax.bitwise_or(lax.shift_right_logical(lo, 16), hi)
    pna_ref[...] = packed
    pnb_ref[...] = packed


def _embed(features, W1x, W1n):
    n, d = features.shape
    h = W1x.shape[1]
    rb = 2000
    assert n % rb == 0
    out = pl.pallas_call(
        _embed_body,
        grid=(n // rb,),
        in_specs=[
            pl.BlockSpec((rb, d), lambda i: (i, 0)),
            pl.BlockSpec((d, h), lambda i: (0, 0)),
            pl.BlockSpec((d, h), lambda i: (0, 0)),
        ],
        out_specs=[
            pl.BlockSpec((rb, h), lambda i: (i, 0)),
            pl.BlockSpec((rb, h), lambda i: (i, 0)),
            pl.BlockSpec((rb, h // 2), lambda i: (i, 0)),
            pl.BlockSpec((rb, h // 2), lambda i: (i, 0)),
        ],
        out_shape=[
            jax.ShapeDtypeStruct((n, h), jnp.float32),
            jax.ShapeDtypeStruct((n, h), jnp.float32),
            jax.ShapeDtypeStruct((n, h // 2), jnp.int32),
            jax.ShapeDtypeStruct((n, h // 2), jnp.int32),
        ],
    )(features, W1x, W1n)
    return out


def _mean10(PN, PNb, nbr10):
    hi32 = PN.shape[1]
    h = hi32 * 2
    npad = nbr10.shape[0] * nbr10.shape[1] // 10
    g = 8
    nbuf = 4
    npw0 = 384
    npw1 = (npad - 16 * npw0) // 16
    nblk0, nblk1 = npw0 // g, npw1 // g
    assert nblk0 % nbuf == 0 and nblk1 % nbuf == 0
    nblk_max = max(nblk0, nblk1)

    @functools.partial(
        pl.kernel,
        mesh=_sc_mesh(),
        out_type=jax.ShapeDtypeStruct((npad, h), jnp.float32),
        scratch_types=[
            pltpu.VMEM((nblk_max, g * 10), jnp.int32),
            [pltpu.VMEM((g * 10, hi32), jnp.int32)] * nbuf,
            [pltpu.VMEM((g, h), jnp.float32)] * nbuf,
            [pltpu.SemaphoreType.DMA] * nbuf,
            [pltpu.SemaphoreType.DMA] * nbuf,
        ],
    )
    def k(pn_hbm, pnb_hbm, idx_hbm, out_hbm, idxall_v, rows_v, acc_v, gsem, wsem):
        core = lax.axis_index("c")
        sid = lax.axis_index("s")
        w0 = pl.multiple_of(
            jnp.where(core == 0, sid * npw0, 16 * npw0 + sid * npw1), 8)
        nblk_w = jnp.where(core == 0, nblk0, nblk1)
        nouter_w = jnp.where(core == 0, nblk0 // nbuf, nblk1 // nbuf)

        def gather(q, t):
            return pltpu.make_async_copy(
                pn_hbm.at[idxall_v.at[q]], rows_v[t], gsem[t])

        def gather_start(q, t):
            pl.when(core == 0)(lambda: gather(q, t).start())
            pl.when(core == 1)(lambda: pltpu.make_async_copy(
                pnb_hbm.at[idxall_v.at[q]], rows_v[t], gsem[t]).start())

        def wback(q, t):
            return pltpu.make_async_copy(
                acc_v[t], out_hbm.at[pl.ds(w0 + q * g, g)], wsem[t])

        pl.when(core == 0)(lambda: pltpu.sync_copy(
            idx_hbm.at[pl.ds(sid * nblk0, nblk0)], idxall_v))
        pl.when(core == 1)(lambda: pltpu.sync_copy(
            idx_hbm.at[pl.ds(16 * nblk0 + sid * nblk1, nblk1)],
            idxall_v.at[pl.ds(0, nblk1)]))
        for t in range(nbuf):
            gather_start(t, t)

        def outer(bb, carry):
            for t in range(nbuf):
                q = bb * nbuf + t
                gather(q, t).wait()
                pl.when(q >= nbuf)(lambda: wback(q - nbuf, t).wait())

                def node(i, carry2):
                    r0 = i * 10
                    for c in range(hi32 // _LANES):
                        s = pl.ds(c * _LANES, _LANES)
                        a = jnp.zeros((_LANES,), jnp.float32)
                        b2 = jnp.zeros((_LANES,), jnp.float32)
                        for kk in range(10):
                            v = rows_v[t][r0 + kk, s]
                            a = a + lax.bitcast_convert_type(
                                lax.shift_left(v, 16), jnp.float32)
                            b2 = b2 + lax.bitcast_convert_type(
                                v & jnp.int32(-65536), jnp.float32)
                        acc_v[t][i, s] = jnp.maximum(a * 0.1, 0.0)
                        acc_v[t][i, pl.ds(hi32 + c * _LANES, _LANES)] = (
                            jnp.maximum(b2 * 0.1, 0.0))
                    return carry2

                lax.fori_loop(0, g, node, 0)
                wback(q, t).start()
                pl.when(q + nbuf < nblk_w)(lambda: gather_start(q + nbuf, t))
            return carry

        lax.fori_loop(0, nouter_w, outer, 0)
        for t in range(nbuf):
            wback(nblk_w - nbuf + t, t).wait()

    return k(PN, PNb, nbr10)


def _seeds(RX, PN, RM, ids, nbr25p):
    h = RX.shape[1]
    b = ids.shape[0]
    spw = b // _NW
    g = 4
    nblk = spw // g
    rows = g * 25
    stride = 104
    nc = h // _LANES
    assert nbr25p.shape == (_NW * nblk, stride)

    @functools.partial(
        pl.kernel,
        mesh=_sc_mesh(),
        out_type=[jax.ShapeDtypeStruct((b, h), jnp.float32)] * 4,
        scratch_types=[
            pltpu.VMEM((spw,), jnp.int32),
            pltpu.VMEM((nblk, stride), jnp.int32),
            [pltpu.VMEM((stride, h), jnp.float32)] * 3,
            pltpu.VMEM((spw, h), jnp.float32),
            [pltpu.VMEM((g, h), jnp.float32)] * 3,
            [pltpu.SemaphoreType.DMA] * 3,
            [pltpu.SemaphoreType.DMA] * 3,
            pltpu.SemaphoreType.DMA,
        ],
    )
    def k(rx_hbm, pn_hbm, rm_hbm, ids_hbm, nbr_hbm,
          x0_hbm, m25_hbm, aga_hbm, agb_hbm,
          ids_v, idxall_v, rows_v, x0_v, acc_v, gsem, wsem, xsem):
        s0 = _wid() * spw
        tables = [pn_hbm, rx_hbm, rm_hbm]
        outs = [m25_hbm, aga_hbm, agb_hbm]
        relus = [True, False, False]

        def gather(bb, t):
            return pltpu.make_async_copy(
                tables[t].at[idxall_v.at[bb]], rows_v[t], gsem[t])

        def wback(bb, t):
            return pltpu.make_async_copy(
                acc_v[t], outs[t].at[pl.ds(s0 + bb * g, g)], wsem[t])

        pltpu.sync_copy(ids_hbm.at[pl.ds(s0, spw)], ids_v)
        x0copy = pltpu.make_async_copy(rx_hbm.at[ids_v], x0_v, xsem)
        x0copy.start()
        pltpu.sync_copy(nbr_hbm.at[pl.ds(_wid() * nblk, nblk)], idxall_v)
        for t in range(3):
            gather(0, t).start()

        def blk(bb, carry):
            for t in range(3):
                gather(bb, t).wait()
                pl.when(bb >= 1)(lambda: wback(bb - 1, t).wait())

                def node(i, carry2):
                    r0 = i * 25
                    for c in range(nc):
                        s = pl.ds(c * _LANES, _LANES)
                        a = rows_v[t][r0, s]
                        for kk in range(1, 25):
                            a = a + rows_v[t][r0 + kk, s]
                        a = a * (1.0 / 25.0)
                        if relus[t]:
                            a = jnp.maximum(a, 0.0)
                        acc_v[t][i, s] = a
                    return carry2

                lax.fori_loop(0, g, node, 0)
                wback(bb, t).start()
                pl.when(bb + 1 < nblk)(lambda: gather(bb + 1, t).start())
            return carry

        lax.fori_loop(0, nblk, blk, 0)
        x0copy.wait()
        pltpu.sync_copy(x0_v, x0_hbm.at[pl.ds(s0, spw)])
        for t in range(3):
            wback(nblk - 1, t).wait()

    return k(RX, PN, RM, ids, nbr25p)


def _head_body(x0a_ref, x0b_ref, a2a_ref, a2b_ref, w2x_ref, w2n_ref,
               fcw_ref, fcb_ref, o_ref):
    x0 = jnp.concatenate([x0a_ref[...], x0b_ref[...]], axis=1)
    a2 = jnp.concatenate([a2a_ref[...], a2b_ref[...]], axis=1)
    hx = jnp.dot(x0, w2x_ref[...], preferred_element_type=jnp.float32)
    hn = jnp.dot(a2, w2n_ref[...], preferred_element_type=jnp.float32)
    hcat = jnp.concatenate([hx, hn], axis=1)
    ss = jnp.sum(hcat * hcat, axis=1, keepdims=True)
    nrm = jnp.maximum(jnp.sqrt(ss), 1e-12)
    hcat = hcat / nrm
    o_ref[...] = (
        jnp.dot(hcat, fcw_ref[...], preferred_element_type=jnp.float32)
        + fcb_ref[...]
    )


def _head(X0, RM25, AGa, AGb, W2x, W2n, fcW, fcb):
    b = X0.shape[0]
    ncls = fcW.shape[1]
    return pl.pallas_call(
        _head_body,
        out_shape=jax.ShapeDtypeStruct((b, ncls), jnp.float32),
    )(X0, RM25, AGa, AGb, W2x, W2n, fcW, fcb.reshape(1, -1))


def kernel(ids, features, adj, W1x, W1n, W2x, W2n, fcW, fcb):
    ids = ids.astype(jnp.int32)
    adj = adj.astype(jnp.int32)
    n = features.shape[0]
    npad = ((n + 8 * _NW - 1) // (8 * _NW)) * (8 * _NW)

    RX, PN, PNia, PNib = _embed(features, W1x, W1n)
    nbr10 = jnp.pad(adj[:, :10], ((0, npad - n), (0, 0))).reshape(-1, 80)
    RM = _mean10(PNia, PNib, nbr10)
    nbr25 = jnp.take(adj, ids, axis=0)[:, :25].reshape(-1, 100)
    nbr25p = jnp.pad(nbr25, ((0, 0), (0, 4)))
    X0, RM25, AGa, AGb = _seeds(RX, PN, RM, ids, nbr25p)
    return _head(X0, RM25, AGa, AGb, W2x, W2n, fcW, fcb)
